# initial kernel scaffold (unmeasured)
import jax
import jax.numpy as jnp
from jax import lax
from jax.experimental import pallas as pl
from jax.experimental.pallas import tpu as pltpu

N_DEV = 16


def kernel(x, w_mat):
    k_total, k_per = x.shape
    m_per = k_total // N_DEV
    n = w_mat.shape[1]

    def body(x_ref, w_ref, out_ref, xg_ref, comm_ref, send_sems, recv_sems):
        my = lax.axis_index("i")

        rdmas = []
        for d in range(1, N_DEV):
            tgt = lax.rem(my + d, N_DEV)
            rdma = pltpu.make_async_remote_copy(
                src_ref=x_ref.at[pl.ds(tgt * m_per, m_per), :],
                dst_ref=comm_ref.at[d - 1],
                send_sem=send_sems.at[d - 1],
                recv_sem=recv_sems.at[d - 1],
                device_id=(tgt,),
                device_id_type=pl.DeviceIdType.MESH,
            )
            rdma.start()
            rdmas.append(rdma)

        xg_ref[:, pl.ds(my * k_per, k_per)] = x_ref[pl.ds(my * m_per, m_per), :]

        for d in range(1, N_DEV):
            rdmas[d - 1].wait_recv()
            src = lax.rem(my - d + N_DEV, N_DEV)
            xg_ref[:, pl.ds(src * k_per, k_per)] = comm_ref[d - 1]

        out_ref[:, :] = jnp.dot(
            xg_ref[:, :], w_ref[:, :], preferred_element_type=jnp.float32
        )

        for d in range(1, N_DEV):
            rdmas[d - 1].wait_send()

    return pl.pallas_call(
        body,
        out_shape=jax.ShapeDtypeStruct((m_per, n), jnp.float32),
        in_specs=[
            pl.BlockSpec(memory_space=pltpu.VMEM),
            pl.BlockSpec(memory_space=pltpu.VMEM),
        ],
        out_specs=pl.BlockSpec(memory_space=pltpu.VMEM),
        scratch_shapes=[
            pltpu.VMEM((m_per, k_total), x.dtype),
            pltpu.VMEM((N_DEV - 1, m_per, k_per), x.dtype),
            pltpu.SemaphoreType.DMA((N_DEV - 1,)),
            pltpu.SemaphoreType.DMA((N_DEV - 1,)),
        ],
    )(x, w_mat)


# baseline (device time: 19974 ns/iter reference)
import jax
import jax.numpy as jnp
from jax import lax
from jax.experimental import pallas as pl
from jax.experimental.pallas import tpu as pltpu

N_DEV = 16


def kernel(x, w_mat):
    k_total, k_per = x.shape
    m_per = k_total // N_DEV
    n = w_mat.shape[1]

    def body(x_ref, w_ref, out_ref, comm_ref, send_sems, recv_sems):
        my = lax.axis_index("i")

        rdmas = []
        for d in range(1, N_DEV):
            tgt = lax.rem(my + d, N_DEV)
            rdma = pltpu.make_async_remote_copy(
                src_ref=x_ref.at[pl.ds(tgt * m_per, m_per), :],
                dst_ref=comm_ref.at[d - 1],
                send_sem=send_sems.at[d - 1],
                recv_sem=recv_sems.at[d - 1],
                device_id=(tgt,),
                device_id_type=pl.DeviceIdType.MESH,
            )
            rdma.start()
            rdmas.append(rdma)

        out_ref[:, :] = jnp.dot(
            x_ref[pl.ds(my * m_per, m_per), :],
            w_ref[pl.ds(my * k_per, k_per), :],
            preferred_element_type=jnp.float32,
        )

        for d in range(1, N_DEV):
            rdmas[d - 1].wait_recv()
            src = lax.rem(my - d + N_DEV, N_DEV)
            out_ref[:, :] += jnp.dot(
                comm_ref[d - 1],
                w_ref[pl.ds(src * k_per, k_per), :],
                preferred_element_type=jnp.float32,
            )

        for d in range(1, N_DEV):
            rdmas[d - 1].wait_send()

    return pl.pallas_call(
        body,
        out_shape=jax.ShapeDtypeStruct((m_per, n), jnp.float32),
        in_specs=[
            pl.BlockSpec(memory_space=pltpu.VMEM),
            pl.BlockSpec(memory_space=pltpu.VMEM),
        ],
        out_specs=pl.BlockSpec(memory_space=pltpu.VMEM),
        scratch_shapes=[
            pltpu.VMEM((N_DEV - 1, m_per, k_per), x.dtype),
            pltpu.SemaphoreType.DMA((N_DEV - 1,)),
            pltpu.SemaphoreType.DMA((N_DEV - 1,)),
        ],
    )(x, w_mat)


# device time: 19788 ns/iter; 1.0094x vs baseline; 1.0094x over previous
import jax
import jax.numpy as jnp
from jax import lax
from jax.experimental import pallas as pl
from jax.experimental.pallas import tpu as pltpu

N_DEV = 16


def kernel(x, w_mat):
    k_total, k_per = x.shape
    m_per = k_total // N_DEV
    n = w_mat.shape[1]

    def body(x_ref, w_ref, out_ref, comm_ref, send_sems, recv_sems):
        my = lax.axis_index("i")

        rdmas = []
        for d in range(1, N_DEV):
            tgt = lax.rem(my + d, N_DEV)
            rdma = pltpu.make_async_remote_copy(
                src_ref=x_ref.at[pl.ds(tgt * m_per, m_per), :],
                dst_ref=comm_ref.at[my],
                send_sem=send_sems.at[d - 1],
                recv_sem=recv_sems.at[my],
                device_id=(tgt,),
                device_id_type=pl.DeviceIdType.MESH,
            )
            rdma.start()
            rdmas.append(rdma)

        comm_ref[my] = x_ref[pl.ds(my * m_per, m_per), :]

        for s in range(N_DEV):
            @pl.when(s != my)
            def _():
                recv = pltpu.make_async_remote_copy(
                    src_ref=x_ref.at[pl.ds(0, m_per), :],
                    dst_ref=comm_ref.at[s],
                    send_sem=send_sems.at[0],
                    recv_sem=recv_sems.at[s],
                    device_id=(my,),
                    device_id_type=pl.DeviceIdType.MESH,
                )
                recv.wait_recv()

        xg = jnp.concatenate([comm_ref[s] for s in range(N_DEV)], axis=1)
        out_ref[:, :] = jnp.dot(xg, w_ref[:, :], preferred_element_type=jnp.float32)

        for d in range(1, N_DEV):
            rdmas[d - 1].wait_send()

    return pl.pallas_call(
        body,
        out_shape=jax.ShapeDtypeStruct((m_per, n), jnp.float32),
        in_specs=[
            pl.BlockSpec(memory_space=pltpu.VMEM),
            pl.BlockSpec(memory_space=pltpu.VMEM),
        ],
        out_specs=pl.BlockSpec(memory_space=pltpu.VMEM),
        scratch_shapes=[
            pltpu.VMEM((N_DEV, m_per, k_per), x.dtype),
            pltpu.SemaphoreType.DMA((N_DEV - 1,)),
            pltpu.SemaphoreType.DMA((N_DEV,)),
        ],
    )(x, w_mat)


# device time: 16324 ns/iter; 1.2236x vs baseline; 1.2122x over previous
import jax
import jax.numpy as jnp
from jax import lax
from jax.experimental import pallas as pl
from jax.experimental.pallas import tpu as pltpu

N_DEV = 16


def kernel(x, w_mat):
    k_total, k_per = x.shape
    m_per = k_total // N_DEV
    n = w_mat.shape[1]

    def body(x_ref, w_hbm_ref, out_ref, w_ref, comm_ref, send_sems, recv_sems,
             w_sem):
        my = lax.axis_index("i")

        w_load = pltpu.make_async_copy(w_hbm_ref, w_ref, w_sem)
        w_load.start()

        comm_ref[my] = x_ref[pl.ds(my * m_per, m_per), :]

        barrier_sem = pltpu.get_barrier_semaphore()
        for d in range(1, N_DEV):
            pl.semaphore_signal(
                barrier_sem, inc=1,
                device_id=(lax.rem(my + d, N_DEV),),
                device_id_type=pl.DeviceIdType.MESH,
            )
        pl.semaphore_wait(barrier_sem, N_DEV - 1)

        rdmas = []
        for d in range(1, N_DEV):
            tgt = lax.rem(my + d, N_DEV)
            rdma = pltpu.make_async_remote_copy(
                src_ref=x_ref.at[pl.ds(tgt * m_per, m_per), :],
                dst_ref=comm_ref.at[my],
                send_sem=send_sems.at[d - 1],
                recv_sem=recv_sems.at[my],
                device_id=(tgt,),
                device_id_type=pl.DeviceIdType.MESH,
            )
            rdma.start()
            rdmas.append(rdma)

        for s in range(N_DEV):
            @pl.when(s != my)
            def _():
                recv = pltpu.make_async_remote_copy(
                    src_ref=x_ref.at[pl.ds(0, m_per), :],
                    dst_ref=comm_ref.at[s],
                    send_sem=send_sems.at[0],
                    recv_sem=recv_sems.at[s],
                    device_id=(my,),
                    device_id_type=pl.DeviceIdType.MESH,
                )
                recv.wait_recv()

        w_load.wait()

        xg = jnp.concatenate([comm_ref[s] for s in range(N_DEV)], axis=1)
        out_ref[:, :] = jnp.dot(xg, w_ref[:, :], preferred_element_type=jnp.float32)

        for d in range(1, N_DEV):
            rdmas[d - 1].wait_send()

    return pl.pallas_call(
        body,
        out_shape=jax.ShapeDtypeStruct((m_per, n), jnp.float32),
        in_specs=[
            pl.BlockSpec(memory_space=pltpu.VMEM),
            pl.BlockSpec(memory_space=pl.ANY),
        ],
        out_specs=pl.BlockSpec(memory_space=pltpu.VMEM),
        scratch_shapes=[
            pltpu.VMEM((k_total, n), w_mat.dtype),
            pltpu.VMEM((N_DEV, m_per, k_per), x.dtype),
            pltpu.SemaphoreType.DMA((N_DEV - 1,)),
            pltpu.SemaphoreType.DMA((N_DEV,)),
            pltpu.SemaphoreType.DMA,
        ],
        compiler_params=pltpu.CompilerParams(collective_id=0),
    )(x, w_mat)
